# Initial kernel scaffold; baseline (speedup 1.0000x reference)
import functools

import jax
import jax.numpy as jnp
from jax import lax
from jax.experimental import pallas as pl
from jax.experimental.pallas import tpu as pltpu
from jax.experimental.pallas import tpu_sc as plsc

N = 10000
F = 128
NLABEL = 40
E = 320000
EE = E + N
NW = 32
C = 128
K = 82
CAP = NW * K * C
N_PAD = 10240
NTILE = 16
STRIPE = N_PAD // NTILE
DEGW = 16

_mesh = plsc.VectorSubcoreMesh(core_axis_name="c", subcore_axis_name="s")


def _sc_deg(dst_idx, ones, zdeg):
    @functools.partial(
        pl.kernel,
        out_type=jax.ShapeDtypeStruct((2, N_PAD, DEGW), jnp.float32),
        mesh=_mesh,
        scratch_types=[
            pltpu.VMEM((K, C), jnp.int32),
            pltpu.VMEM((C, DEGW), jnp.float32),
            pltpu.VMEM_SHARED((N_PAD, DEGW), jnp.float32),
        ],
    )
    def body(dst_hbm, ones_hbm, z_hbm, out_hbm, dst_v, ones_v, acc):
        c = lax.axis_index("c")
        s = lax.axis_index("s")
        wid = c * NTILE + s
        pltpu.sync_copy(dst_hbm.at[wid], dst_v)
        pltpu.sync_copy(ones_hbm, ones_v)
        pltpu.sync_copy(z_hbm.at[pl.ds(s * STRIPE, STRIPE)],
                        acc.at[pl.ds(s * STRIPE, STRIPE)])
        plsc.subcore_barrier()

        def step(j, carry):
            pltpu.sync_copy(ones_v, acc.at[dst_v.at[j]], add=True)
            return carry

        lax.fori_loop(0, K, step, 0)
        plsc.subcore_barrier()
        pltpu.sync_copy(acc.at[pl.ds(s * STRIPE, STRIPE)],
                        out_hbm.at[c, pl.ds(s * STRIPE, STRIPE)])

    return body(dst_idx, ones, zdeg)


def _sc_scatter(g, src_idx, dst_idx, zfeat):
    @functools.partial(
        pl.kernel,
        out_type=jax.ShapeDtypeStruct((2, N_PAD, F), jnp.float32),
        mesh=_mesh,
        scratch_types=[
            pltpu.VMEM((K, C), jnp.int32),
            pltpu.VMEM((K, C), jnp.int32),
            pltpu.VMEM((C, F), jnp.float32),
            pltpu.VMEM_SHARED((N_PAD, F), jnp.float32),
            pltpu.SemaphoreType.DMA,
        ],
    )
    def body(g_hbm, src_hbm, dst_hbm, z_hbm, out_hbm,
             src_v, dst_v, rows, acc, sem):
        c = lax.axis_index("c")
        s = lax.axis_index("s")
        wid = c * NTILE + s
        pltpu.sync_copy(src_hbm.at[wid], src_v)
        pltpu.sync_copy(dst_hbm.at[wid], dst_v)
        pltpu.sync_copy(z_hbm.at[pl.ds(s * STRIPE, STRIPE)],
                        acc.at[pl.ds(s * STRIPE, STRIPE)])
        plsc.subcore_barrier()

        def step(j, carry):
            pltpu.async_copy(g_hbm.at[src_v.at[j]], rows, sem).wait()
            pltpu.sync_copy(rows, acc.at[dst_v.at[j]], add=True)
            return carry

        lax.fori_loop(0, K, step, 0)
        plsc.subcore_barrier()
        pltpu.sync_copy(acc.at[pl.ds(s * STRIPE, STRIPE)],
                        out_hbm.at[c, pl.ds(s * STRIPE, STRIPE)])

    return body(g, src_idx, dst_idx, zfeat)


BLK = 512


def _tc_first(degp, x_pad, W1):
    def body(degp_ref, x_ref, w_ref, dinv_ref, g_ref):
        a = degp_ref[...]
        deg = a[0, :, 0:1] + a[1, :, 0:1]
        dinv = jnp.where(deg > 0, lax.rsqrt(deg), 0.0)
        db = jnp.broadcast_to(dinv, (BLK, F))
        dinv_ref[...] = db
        g_ref[...] = db * jnp.dot(x_ref[...], w_ref[...],
                                  preferred_element_type=jnp.float32)

    return pl.pallas_call(
        body,
        grid=(N_PAD // BLK,),
        in_specs=[
            pl.BlockSpec((2, BLK, DEGW), lambda i: (0, i, 0)),
            pl.BlockSpec((BLK, F), lambda i: (i, 0)),
            pl.BlockSpec((F, F), lambda i: (0, 0)),
        ],
        out_specs=[
            pl.BlockSpec((BLK, F), lambda i: (i, 0)),
            pl.BlockSpec((BLK, F), lambda i: (i, 0)),
        ],
        out_shape=[
            jax.ShapeDtypeStruct((N_PAD, F), jnp.float32),
            jax.ShapeDtypeStruct((N_PAD, F), jnp.float32),
        ],
    )(degp, x_pad, W1)


def _tc_mid(p, dinv_b, b, W):
    def body(p_ref, d_ref, b_ref, w_ref, g_ref):
        a = p_ref[...]
        d = d_ref[...]
        h = jnp.maximum(d * (a[0] + a[1]) + b_ref[...], 0.0)
        g_ref[...] = d * jnp.dot(h, w_ref[...],
                                 preferred_element_type=jnp.float32)

    return pl.pallas_call(
        body,
        grid=(N_PAD // BLK,),
        in_specs=[
            pl.BlockSpec((2, BLK, F), lambda i: (0, i, 0)),
            pl.BlockSpec((BLK, F), lambda i: (i, 0)),
            pl.BlockSpec((1, F), lambda i: (0, 0)),
            pl.BlockSpec((F, F), lambda i: (0, 0)),
        ],
        out_specs=pl.BlockSpec((BLK, F), lambda i: (i, 0)),
        out_shape=jax.ShapeDtypeStruct((N_PAD, F), jnp.float32),
    )(p, dinv_b, b.reshape(1, F), W)


BLKF = 1000


def _tc_final(p, dinv_b, b3, M1, mb1, M2, mb2):
    def body(p_ref, d_ref, b_ref, m1_ref, mb1_ref, m2_ref, mb2_ref, y_ref):
        a = p_ref[...]
        h = jnp.maximum(d_ref[...] * (a[0] + a[1]) + b_ref[...], 0.0)
        u = jnp.dot(h, m1_ref[...],
                    preferred_element_type=jnp.float32) + mb1_ref[...]
        u = jnp.where(u > 0, u, jnp.expm1(u))
        y = jnp.dot(u, m2_ref[...],
                    preferred_element_type=jnp.float32) + mb2_ref[...]
        y = y - jnp.max(y, axis=1, keepdims=True)
        y_ref[...] = y - jnp.log(jnp.sum(jnp.exp(y), axis=1, keepdims=True))

    return pl.pallas_call(
        body,
        grid=(N // BLKF,),
        in_specs=[
            pl.BlockSpec((2, BLKF, F), lambda i: (0, i, 0)),
            pl.BlockSpec((BLKF, F), lambda i: (i, 0)),
            pl.BlockSpec((1, F), lambda i: (0, 0)),
            pl.BlockSpec((F, 2 * F), lambda i: (0, 0)),
            pl.BlockSpec((1, 2 * F), lambda i: (0, 0)),
            pl.BlockSpec((2 * F, NLABEL), lambda i: (0, 0)),
            pl.BlockSpec((1, NLABEL), lambda i: (0, 0)),
        ],
        out_specs=pl.BlockSpec((BLKF, NLABEL), lambda i: (i, 0)),
        out_shape=jax.ShapeDtypeStruct((N, NLABEL), jnp.float32),
    )(p, dinv_b, b3.reshape(1, F), M1, mb1.reshape(1, 2 * F),
      M2, mb2.reshape(1, NLABEL))


def kernel(x, adj, W1, b1, W2, b2, W3, b3, M1, mb1, M2, mb2):
    loops = jnp.arange(N, dtype=jnp.int32)
    src = jnp.concatenate([adj[0].astype(jnp.int32), loops])
    dst = jnp.concatenate([adj[1].astype(jnp.int32), loops])
    fill = jnp.full((CAP - EE,), N, jnp.int32)
    src_idx = jnp.concatenate([src, fill]).reshape(NW, K, C)
    dst_idx = jnp.concatenate([dst, fill]).reshape(NW, K, C)
    x_pad = jnp.zeros((N_PAD, F), jnp.float32).at[:N].set(x)
    ones = jnp.ones((C, DEGW), jnp.float32)
    zdeg = jnp.zeros((N_PAD, DEGW), jnp.float32)
    zfeat = jnp.zeros((N_PAD, F), jnp.float32)

    degp = _sc_deg(dst_idx, ones, zdeg)
    dinv_b, g = _tc_first(degp, x_pad, W1)
    p = _sc_scatter(g, src_idx, dst_idx, zfeat)
    g = _tc_mid(p, dinv_b, b1, W2)
    p = _sc_scatter(g, src_idx, dst_idx, zfeat)
    g = _tc_mid(p, dinv_b, b2, W3)
    p = _sc_scatter(g, src_idx, dst_idx, zfeat)
    return _tc_final(p, dinv_b, b3, M1, mb1, M2, mb2)


# speedup vs baseline: 8.5478x; 8.5478x over previous
import functools

import jax
import jax.numpy as jnp
from jax import lax
from jax.experimental import pallas as pl
from jax.experimental.pallas import tpu as pltpu
from jax.experimental.pallas import tpu_sc as plsc

N = 10000
F = 128
NLABEL = 40
E = 320000
EE = E + N
NW = 32
C = 128
K = 82
CAP = NW * K * C
N_PAD = 10240
NTILE = 16
STRIPE = N_PAD // NTILE
DEGW = 128

_mesh = plsc.VectorSubcoreMesh(core_axis_name="c", subcore_axis_name="s")


def _sc_deg(dst_idx, ones, zdeg):
    @functools.partial(
        pl.kernel,
        out_type=jax.ShapeDtypeStruct((2, N_PAD, DEGW), jnp.float32),
        mesh=_mesh,
        scratch_types=[
            pltpu.VMEM((K, C), jnp.int32),
            pltpu.VMEM((C, DEGW), jnp.float32),
            pltpu.VMEM_SHARED((N_PAD, DEGW), jnp.float32),
        ],
    )
    def body(dst_hbm, ones_hbm, z_hbm, out_hbm, dst_v, ones_v, acc):
        c = lax.axis_index("c")
        s = lax.axis_index("s")
        wid = c * NTILE + s
        pltpu.sync_copy(dst_hbm.at[wid], dst_v)
        pltpu.sync_copy(ones_hbm, ones_v)
        pltpu.sync_copy(z_hbm.at[pl.ds(s * STRIPE, STRIPE)],
                        acc.at[pl.ds(s * STRIPE, STRIPE)])
        plsc.subcore_barrier()

        def step(j, carry):
            pltpu.sync_copy(ones_v, acc.at[dst_v.at[j]], add=True)
            return carry

        lax.fori_loop(0, K, step, 0)
        plsc.subcore_barrier()
        pltpu.sync_copy(acc.at[pl.ds(s * STRIPE, STRIPE)],
                        out_hbm.at[c, pl.ds(s * STRIPE, STRIPE)])

    return body(dst_idx, ones, zdeg)


def _sc_scatter(g, src_idx, dst_idx, zfeat):
    @functools.partial(
        pl.kernel,
        out_type=jax.ShapeDtypeStruct((2, N_PAD, F), jnp.float32),
        mesh=_mesh,
        scratch_types=[
            pltpu.VMEM((K, C), jnp.int32),
            pltpu.VMEM((K, C), jnp.int32),
            pltpu.VMEM((C, F), jnp.float32),
            pltpu.VMEM_SHARED((N_PAD, F), jnp.float32),
            pltpu.SemaphoreType.DMA,
        ],
    )
    def body(g_hbm, src_hbm, dst_hbm, z_hbm, out_hbm,
             src_v, dst_v, rows, acc, sem):
        c = lax.axis_index("c")
        s = lax.axis_index("s")
        wid = c * NTILE + s
        pltpu.sync_copy(src_hbm.at[wid], src_v)
        pltpu.sync_copy(dst_hbm.at[wid], dst_v)
        pltpu.sync_copy(z_hbm.at[pl.ds(s * STRIPE, STRIPE)],
                        acc.at[pl.ds(s * STRIPE, STRIPE)])
        plsc.subcore_barrier()

        def step(j, carry):
            pltpu.async_copy(g_hbm.at[src_v.at[j]], rows, sem).wait()
            pltpu.sync_copy(rows, acc.at[dst_v.at[j]], add=True)
            return carry

        lax.fori_loop(0, K, step, 0)
        plsc.subcore_barrier()
        pltpu.sync_copy(acc.at[pl.ds(s * STRIPE, STRIPE)],
                        out_hbm.at[c, pl.ds(s * STRIPE, STRIPE)])

    return body(g, src_idx, dst_idx, zfeat)


BLK = 512


def _tc_first(degp, x_pad, W1):
    def body(degp_ref, x_ref, w_ref, dinv_ref, g_ref):
        a = degp_ref[...]
        deg = a[0, :, 0:1] + a[1, :, 0:1]
        dinv = jnp.where(deg > 0, 1.0 / jnp.sqrt(jnp.maximum(deg, 1.0)), 0.0)
        db = jnp.broadcast_to(dinv, (BLK, F))
        dinv_ref[...] = db
        g_ref[...] = db * jnp.dot(x_ref[...], w_ref[...],
                                  preferred_element_type=jnp.float32,
                                  precision=lax.Precision.HIGHEST)

    return pl.pallas_call(
        body,
        grid=(N_PAD // BLK,),
        in_specs=[
            pl.BlockSpec((2, BLK, DEGW), lambda i: (0, i, 0)),
            pl.BlockSpec((BLK, F), lambda i: (i, 0)),
            pl.BlockSpec((F, F), lambda i: (0, 0)),
        ],
        out_specs=[
            pl.BlockSpec((BLK, F), lambda i: (i, 0)),
            pl.BlockSpec((BLK, F), lambda i: (i, 0)),
        ],
        out_shape=[
            jax.ShapeDtypeStruct((N_PAD, F), jnp.float32),
            jax.ShapeDtypeStruct((N_PAD, F), jnp.float32),
        ],
    )(degp, x_pad, W1)


def _tc_mid(p, dinv_b, b, W):
    def body(p_ref, d_ref, b_ref, w_ref, g_ref):
        a = p_ref[...]
        d = d_ref[...]
        h = jnp.maximum(d * (a[0] + a[1]) + b_ref[...], 0.0)
        g_ref[...] = d * jnp.dot(h, w_ref[...],
                                 preferred_element_type=jnp.float32,
                                 precision=lax.Precision.HIGHEST)

    return pl.pallas_call(
        body,
        grid=(N_PAD // BLK,),
        in_specs=[
            pl.BlockSpec((2, BLK, F), lambda i: (0, i, 0)),
            pl.BlockSpec((BLK, F), lambda i: (i, 0)),
            pl.BlockSpec((1, F), lambda i: (0, 0)),
            pl.BlockSpec((F, F), lambda i: (0, 0)),
        ],
        out_specs=pl.BlockSpec((BLK, F), lambda i: (i, 0)),
        out_shape=jax.ShapeDtypeStruct((N_PAD, F), jnp.float32),
    )(p, dinv_b, b.reshape(1, F), W)


BLKF = 1000


def _tc_final(p, dinv_b, b3, M1, mb1, M2, mb2):
    def body(p_ref, d_ref, b_ref, m1_ref, mb1_ref, m2_ref, mb2_ref, y_ref):
        a = p_ref[...]
        h = jnp.maximum(d_ref[...] * (a[0] + a[1]) + b_ref[...], 0.0)
        u = jnp.dot(h, m1_ref[...],
                    preferred_element_type=jnp.float32,
                    precision=lax.Precision.HIGHEST) + mb1_ref[...]
        u = jnp.where(u > 0, u, jnp.exp(jnp.minimum(u, 0.0)) - 1.0)
        y = jnp.dot(u, m2_ref[...],
                    preferred_element_type=jnp.float32,
                    precision=lax.Precision.HIGHEST) + mb2_ref[...]
        y = y - jnp.max(y, axis=1, keepdims=True)
        y_ref[...] = y - jnp.log(jnp.sum(jnp.exp(y), axis=1, keepdims=True))

    return pl.pallas_call(
        body,
        grid=(N // BLKF,),
        in_specs=[
            pl.BlockSpec((2, BLKF, F), lambda i: (0, i, 0)),
            pl.BlockSpec((BLKF, F), lambda i: (i, 0)),
            pl.BlockSpec((1, F), lambda i: (0, 0)),
            pl.BlockSpec((F, 2 * F), lambda i: (0, 0)),
            pl.BlockSpec((1, 2 * F), lambda i: (0, 0)),
            pl.BlockSpec((2 * F, NLABEL), lambda i: (0, 0)),
            pl.BlockSpec((1, NLABEL), lambda i: (0, 0)),
        ],
        out_specs=pl.BlockSpec((BLKF, NLABEL), lambda i: (i, 0)),
        out_shape=jax.ShapeDtypeStruct((N, NLABEL), jnp.float32),
    )(p, dinv_b, b3.reshape(1, F), M1, mb1.reshape(1, 2 * F),
      M2, mb2.reshape(1, NLABEL))


def kernel(x, adj, W1, b1, W2, b2, W3, b3, M1, mb1, M2, mb2):
    loops = jnp.arange(N, dtype=jnp.int32)
    src = jnp.concatenate([adj[0].astype(jnp.int32), loops])
    dst = jnp.concatenate([adj[1].astype(jnp.int32), loops])
    fill = jnp.full((CAP - EE,), N, jnp.int32)
    src_idx = jnp.concatenate([src, fill]).reshape(NW, K, C)
    dst_idx = jnp.concatenate([dst, fill]).reshape(NW, K, C)
    x_pad = jnp.zeros((N_PAD, F), jnp.float32).at[:N].set(x)
    ones = jnp.ones((C, DEGW), jnp.float32)
    zdeg = jnp.zeros((N_PAD, DEGW), jnp.float32)
    zfeat = jnp.zeros((N_PAD, F), jnp.float32)

    degp = _sc_deg(dst_idx, ones, zdeg)
    dinv_b, g = _tc_first(degp, x_pad, W1)
    p = _sc_scatter(g, src_idx, dst_idx, zfeat)
    g = _tc_mid(p, dinv_b, b1, W2)
    p = _sc_scatter(g, src_idx, dst_idx, zfeat)
    g = _tc_mid(p, dinv_b, b2, W3)
    p = _sc_scatter(g, src_idx, dst_idx, zfeat)
    return _tc_final(p, dinv_b, b3, M1, mb1, M2, mb2)


# speedup vs baseline: 21.2666x; 2.4880x over previous
import functools

import jax
import jax.numpy as jnp
from jax import lax
from jax.experimental import pallas as pl
from jax.experimental.pallas import tpu as pltpu
from jax.experimental.pallas import tpu_sc as plsc

N = 10000
F = 128
NLABEL = 40
E = 320000
EE = E + N
NW = 32
C = 128
K = 88
CAP = NW * K * C
N_PAD = 10240
NTILE = 16
STRIPE = N_PAD // NTILE
DEGW = 128

_mesh = plsc.VectorSubcoreMesh(core_axis_name="c", subcore_axis_name="s")


def _sc_deg(dst_idx, ones, zdeg):
    @functools.partial(
        pl.kernel,
        out_type=jax.ShapeDtypeStruct((2, N_PAD, DEGW), jnp.float32),
        mesh=_mesh,
        scratch_types=[
            pltpu.VMEM((K, C), jnp.int32),
            pltpu.VMEM((C, DEGW), jnp.float32),
            pltpu.VMEM_SHARED((N_PAD, DEGW), jnp.float32),
        ],
    )
    def body(dst_hbm, ones_hbm, z_hbm, out_hbm, dst_v, ones_v, acc):
        c = lax.axis_index("c")
        s = lax.axis_index("s")
        wid = c * NTILE + s
        pltpu.sync_copy(dst_hbm.at[wid], dst_v)
        pltpu.sync_copy(ones_hbm, ones_v)
        pltpu.sync_copy(z_hbm.at[pl.ds(s * STRIPE, STRIPE)],
                        acc.at[pl.ds(s * STRIPE, STRIPE)])
        plsc.subcore_barrier()

        def step(j, carry):
            pltpu.sync_copy(ones_v, acc.at[dst_v.at[j]], add=True)
            return carry

        lax.fori_loop(0, K, step, 0)
        plsc.subcore_barrier()
        pltpu.sync_copy(acc.at[pl.ds(s * STRIPE, STRIPE)],
                        out_hbm.at[c, pl.ds(s * STRIPE, STRIPE)])

    return body(dst_idx, ones, zdeg)


def _sc_scatter(g, src_idx, dst_idx, zfeat):
    @functools.partial(
        pl.kernel,
        out_type=jax.ShapeDtypeStruct((2, N_PAD, F), jnp.float32),
        mesh=_mesh,
        scratch_types=[
            pltpu.VMEM((2, 4, C), jnp.int32),
            pltpu.VMEM((2, 4, C), jnp.int32),
            pltpu.VMEM((C, F), jnp.float32),
            pltpu.VMEM((C, F), jnp.float32),
            pltpu.VMEM_SHARED((N_PAD, F), jnp.float32),
            pltpu.SemaphoreType.DMA,
            pltpu.SemaphoreType.DMA,
            pltpu.SemaphoreType.DMA,
            pltpu.SemaphoreType.DMA,
        ],
    )
    def body(g_hbm, src_hbm, dst_hbm, z_hbm, out_hbm,
             swin, dwin, rows_a, rows_b, acc, sem_a, sem_b, sem_w0, sem_w1):
        c = lax.axis_index("c")
        s = lax.axis_index("s")
        wid = c * NTILE + s
        pltpu.sync_copy(src_hbm.at[wid, pl.ds(0, 4)], swin.at[0])
        pltpu.sync_copy(dst_hbm.at[wid, pl.ds(0, 4)], dwin.at[0])
        pltpu.sync_copy(src_hbm.at[wid, pl.ds(4, 4)], swin.at[1])
        pltpu.sync_copy(dst_hbm.at[wid, pl.ds(4, 4)], dwin.at[1])
        pltpu.async_copy(g_hbm.at[swin.at[0, 0]], rows_a, sem_a)
        pltpu.sync_copy(z_hbm.at[pl.ds(s * STRIPE, STRIPE)],
                        acc.at[pl.ds(s * STRIPE, STRIPE)])
        plsc.subcore_barrier()

        NB = K // 8

        def body8(i, carry):
            base = 8 * i
            for t in range(8):
                b, r = t // 4, t % 4
                rows, sem = (rows_a, sem_a) if t % 2 == 0 else (rows_b, sem_b)
                nrows, nsem = (rows_b, sem_b) if t % 2 == 0 else (rows_a,
                                                                  sem_a)
                if t == 3:
                    @pl.when(i > 0)
                    def _():
                        pltpu.make_async_copy(
                            src_hbm.at[wid, pl.ds(0, 4)], swin.at[1],
                            sem_w1).wait()
                        pltpu.make_async_copy(
                            dst_hbm.at[wid, pl.ds(0, 4)], dwin.at[1],
                            sem_w1).wait()
                if t < 7:
                    nb, nr = (t + 1) // 4, (t + 1) % 4
                    pltpu.async_copy(g_hbm.at[swin.at[nb, nr]], nrows, nsem)
                else:
                    @pl.when(i + 1 < NB)
                    def _():
                        pltpu.make_async_copy(
                            src_hbm.at[wid, pl.ds(0, 4)], swin.at[0],
                            sem_w0).wait()
                        pltpu.make_async_copy(
                            dst_hbm.at[wid, pl.ds(0, 4)], dwin.at[0],
                            sem_w0).wait()
                        pltpu.async_copy(g_hbm.at[swin.at[0, 0]], nrows, nsem)
                pltpu.make_async_copy(g_hbm.at[swin.at[b, r]], rows,
                                      sem).wait()
                pltpu.sync_copy(rows, acc.at[dwin.at[b, r]], add=True)
                if t == 3:
                    @pl.when(i + 1 < NB)
                    def _():
                        pltpu.async_copy(
                            src_hbm.at[wid, pl.ds(base + 8, 4)], swin.at[0],
                            sem_w0)
                        pltpu.async_copy(
                            dst_hbm.at[wid, pl.ds(base + 8, 4)], dwin.at[0],
                            sem_w0)
                if t == 7:
                    @pl.when(i + 1 < NB)
                    def _():
                        pltpu.async_copy(
                            src_hbm.at[wid, pl.ds(base + 12, 4)], swin.at[1],
                            sem_w1)
                        pltpu.async_copy(
                            dst_hbm.at[wid, pl.ds(base + 12, 4)], dwin.at[1],
                            sem_w1)
            return carry

        lax.fori_loop(0, NB, body8, 0)
        plsc.subcore_barrier()
        pltpu.sync_copy(acc.at[pl.ds(s * STRIPE, STRIPE)],
                        out_hbm.at[c, pl.ds(s * STRIPE, STRIPE)])

    return body(g, src_idx, dst_idx, zfeat)


BLK = 512


def _tc_first(degp, x_pad, W1):
    def body(degp_ref, x_ref, w_ref, dinv_ref, g_ref):
        a = degp_ref[...]
        deg = a[0, :, 0:1] + a[1, :, 0:1]
        dinv = jnp.where(deg > 0, 1.0 / jnp.sqrt(jnp.maximum(deg, 1.0)), 0.0)
        db = jnp.broadcast_to(dinv, (BLK, F))
        dinv_ref[...] = db
        g_ref[...] = db * jnp.dot(x_ref[...], w_ref[...],
                                  preferred_element_type=jnp.float32,
                                  precision=lax.Precision.HIGHEST)

    return pl.pallas_call(
        body,
        grid=(N_PAD // BLK,),
        in_specs=[
            pl.BlockSpec((2, BLK, DEGW), lambda i: (0, i, 0)),
            pl.BlockSpec((BLK, F), lambda i: (i, 0)),
            pl.BlockSpec((F, F), lambda i: (0, 0)),
        ],
        out_specs=[
            pl.BlockSpec((BLK, F), lambda i: (i, 0)),
            pl.BlockSpec((BLK, F), lambda i: (i, 0)),
        ],
        out_shape=[
            jax.ShapeDtypeStruct((N_PAD, F), jnp.float32),
            jax.ShapeDtypeStruct((N_PAD, F), jnp.float32),
        ],
    )(degp, x_pad, W1)


def _tc_mid(p, dinv_b, b, W):
    def body(p_ref, d_ref, b_ref, w_ref, g_ref):
        a = p_ref[...]
        d = d_ref[...]
        h = jnp.maximum(d * (a[0] + a[1]) + b_ref[...], 0.0)
        g_ref[...] = d * jnp.dot(h, w_ref[...],
                                 preferred_element_type=jnp.float32,
                                 precision=lax.Precision.HIGHEST)

    return pl.pallas_call(
        body,
        grid=(N_PAD // BLK,),
        in_specs=[
            pl.BlockSpec((2, BLK, F), lambda i: (0, i, 0)),
            pl.BlockSpec((BLK, F), lambda i: (i, 0)),
            pl.BlockSpec((1, F), lambda i: (0, 0)),
            pl.BlockSpec((F, F), lambda i: (0, 0)),
        ],
        out_specs=pl.BlockSpec((BLK, F), lambda i: (i, 0)),
        out_shape=jax.ShapeDtypeStruct((N_PAD, F), jnp.float32),
    )(p, dinv_b, b.reshape(1, F), W)


BLKF = 1000


def _tc_final(p, dinv_b, b3, M1, mb1, M2, mb2):
    def body(p_ref, d_ref, b_ref, m1_ref, mb1_ref, m2_ref, mb2_ref, y_ref):
        a = p_ref[...]
        h = jnp.maximum(d_ref[...] * (a[0] + a[1]) + b_ref[...], 0.0)
        u = jnp.dot(h, m1_ref[...],
                    preferred_element_type=jnp.float32,
                    precision=lax.Precision.HIGHEST) + mb1_ref[...]
        u = jnp.where(u > 0, u, jnp.exp(jnp.minimum(u, 0.0)) - 1.0)
        y = jnp.dot(u, m2_ref[...],
                    preferred_element_type=jnp.float32,
                    precision=lax.Precision.HIGHEST) + mb2_ref[...]
        y = y - jnp.max(y, axis=1, keepdims=True)
        y_ref[...] = y - jnp.log(jnp.sum(jnp.exp(y), axis=1, keepdims=True))

    return pl.pallas_call(
        body,
        grid=(N // BLKF,),
        in_specs=[
            pl.BlockSpec((2, BLKF, F), lambda i: (0, i, 0)),
            pl.BlockSpec((BLKF, F), lambda i: (i, 0)),
            pl.BlockSpec((1, F), lambda i: (0, 0)),
            pl.BlockSpec((F, 2 * F), lambda i: (0, 0)),
            pl.BlockSpec((1, 2 * F), lambda i: (0, 0)),
            pl.BlockSpec((2 * F, NLABEL), lambda i: (0, 0)),
            pl.BlockSpec((1, NLABEL), lambda i: (0, 0)),
        ],
        out_specs=pl.BlockSpec((BLKF, NLABEL), lambda i: (i, 0)),
        out_shape=jax.ShapeDtypeStruct((N, NLABEL), jnp.float32),
    )(p, dinv_b, b3.reshape(1, F), M1, mb1.reshape(1, 2 * F),
      M2, mb2.reshape(1, NLABEL))


def kernel(x, adj, W1, b1, W2, b2, W3, b3, M1, mb1, M2, mb2):
    loops = jnp.arange(N, dtype=jnp.int32)
    src = jnp.concatenate([adj[0].astype(jnp.int32), loops])
    dst = jnp.concatenate([adj[1].astype(jnp.int32), loops])
    fill = N + (jnp.arange(CAP - EE, dtype=jnp.int32) % (N_PAD - N))
    src_idx = jnp.concatenate([src, fill]).reshape(NW, K, C)
    dst_idx = jnp.concatenate([dst, fill]).reshape(NW, K, C)
    x_pad = jnp.zeros((N_PAD, F), jnp.float32).at[:N].set(x)
    ones = jnp.ones((C, DEGW), jnp.float32)
    zdeg = jnp.zeros((N_PAD, DEGW), jnp.float32)
    zfeat = jnp.zeros((N_PAD, F), jnp.float32)

    degp = _sc_deg(dst_idx, ones, zdeg)
    dinv_b, g = _tc_first(degp, x_pad, W1)
    p = _sc_scatter(g, src_idx, dst_idx, zfeat)
    g = _tc_mid(p, dinv_b, b1, W2)
    p = _sc_scatter(g, src_idx, dst_idx, zfeat)
    g = _tc_mid(p, dinv_b, b2, W3)
    p = _sc_scatter(g, src_idx, dst_idx, zfeat)
    return _tc_final(p, dinv_b, b3, M1, mb1, M2, mb2)


# speedup vs baseline: 22.2688x; 1.0471x over previous
import functools

import jax
import jax.numpy as jnp
from jax import lax
from jax.experimental import pallas as pl
from jax.experimental.pallas import tpu as pltpu
from jax.experimental.pallas import tpu_sc as plsc

N = 10000
F = 128
NLABEL = 40
E = 320000
EE = E + N
NW = 32
C = 128
K = 81
KP = 84
CAP = NW * KP * C
N_PAD = 10240
NTILE = 16
STRIPE = N_PAD // NTILE
DEGW = 128

_mesh = plsc.VectorSubcoreMesh(core_axis_name="c", subcore_axis_name="s")


def _sc_deg(dst_idx, ones, zdeg):
    @functools.partial(
        pl.kernel,
        out_type=jax.ShapeDtypeStruct((2, N_PAD, DEGW), jnp.float32),
        mesh=_mesh,
        scratch_types=[
            pltpu.VMEM((KP, C), jnp.int32),
            pltpu.VMEM((C, DEGW), jnp.float32),
            pltpu.VMEM_SHARED((N_PAD, DEGW), jnp.float32),
            pltpu.SemaphoreType.DMA,
        ],
    )
    def body(dst_hbm, ones_hbm, z_hbm, out_hbm, dst_v, ones_v, acc, sem):
        c = lax.axis_index("c")
        s = lax.axis_index("s")
        wid = c * NTILE + s
        pltpu.sync_copy(dst_hbm.at[wid], dst_v)
        pltpu.sync_copy(ones_hbm, ones_v)
        pltpu.sync_copy(z_hbm.at[pl.ds(s * STRIPE, STRIPE)],
                        acc.at[pl.ds(s * STRIPE, STRIPE)])
        plsc.subcore_barrier()

        def step(j, carry):
            pltpu.sync_copy(ones_v, acc.at[dst_v.at[j]], add=True)
            return carry

        lax.fori_loop(0, K, step, 0)
        plsc.subcore_barrier()
        pltpu.sync_copy(acc.at[pl.ds(s * STRIPE, STRIPE)],
                        out_hbm.at[c, pl.ds(s * STRIPE, STRIPE)])

    return body(dst_idx, ones, zdeg)


def _sc_scatter(g, src_idx, dst_idx, zfeat):
    @functools.partial(
        pl.kernel,
        out_type=jax.ShapeDtypeStruct((2, N_PAD, F), jnp.float32),
        mesh=_mesh,
        scratch_types=[
            pltpu.VMEM((2, 4, C), jnp.int32),
            pltpu.VMEM((2, 4, C), jnp.int32),
            pltpu.VMEM((C, F), jnp.float32),
            pltpu.VMEM((C, F), jnp.float32),
            pltpu.VMEM_SHARED((N_PAD, F), jnp.float32),
            pltpu.SemaphoreType.DMA,
            pltpu.SemaphoreType.DMA,
            pltpu.SemaphoreType.DMA,
            pltpu.SemaphoreType.DMA,
        ],
    )
    def body(g_hbm, src_hbm, dst_hbm, z_hbm, out_hbm,
             swin, dwin, rows_a, rows_b, acc, sem_a, sem_b, sem_w0, sem_w1):
        c = lax.axis_index("c")
        s = lax.axis_index("s")
        wid = c * NTILE + s
        pltpu.sync_copy(src_hbm.at[wid, pl.ds(0, 4)], swin.at[0])
        pltpu.sync_copy(dst_hbm.at[wid, pl.ds(0, 4)], dwin.at[0])
        pltpu.sync_copy(src_hbm.at[wid, pl.ds(4, 4)], swin.at[1])
        pltpu.sync_copy(dst_hbm.at[wid, pl.ds(4, 4)], dwin.at[1])
        pltpu.async_copy(g_hbm.at[swin.at[0, 0]], rows_a, sem_a)
        pltpu.sync_copy(z_hbm.at[pl.ds(s * STRIPE, STRIPE)],
                        acc.at[pl.ds(s * STRIPE, STRIPE)])
        plsc.subcore_barrier()

        NB = (K - 1) // 8

        def body8(i, carry):
            base = 8 * i
            for t in range(8):
                b, r = t // 4, t % 4
                rows, sem = (rows_a, sem_a) if t % 2 == 0 else (rows_b, sem_b)
                nrows, nsem = (rows_b, sem_b) if t % 2 == 0 else (rows_a,
                                                                  sem_a)
                if t == 3:
                    @pl.when(i > 0)
                    def _():
                        pltpu.make_async_copy(
                            src_hbm.at[wid, pl.ds(0, 4)], swin.at[1],
                            sem_w1).wait()
                        pltpu.make_async_copy(
                            dst_hbm.at[wid, pl.ds(0, 4)], dwin.at[1],
                            sem_w1).wait()
                if t < 7:
                    nb, nr = (t + 1) // 4, (t + 1) % 4
                    pltpu.async_copy(g_hbm.at[swin.at[nb, nr]], nrows, nsem)
                else:
                    @pl.when(i + 1 < NB)
                    def _():
                        pltpu.make_async_copy(
                            src_hbm.at[wid, pl.ds(0, 4)], swin.at[0],
                            sem_w0).wait()
                        pltpu.make_async_copy(
                            dst_hbm.at[wid, pl.ds(0, 4)], dwin.at[0],
                            sem_w0).wait()
                        pltpu.async_copy(g_hbm.at[swin.at[0, 0]], nrows, nsem)
                pltpu.make_async_copy(g_hbm.at[swin.at[b, r]], rows,
                                      sem).wait()
                pltpu.sync_copy(rows, acc.at[dwin.at[b, r]], add=True)
                if t == 3:
                    @pl.when(i + 1 < NB)
                    def _():
                        pltpu.async_copy(
                            src_hbm.at[wid, pl.ds(base + 8, 4)], swin.at[0],
                            sem_w0)
                        pltpu.async_copy(
                            dst_hbm.at[wid, pl.ds(base + 8, 4)], dwin.at[0],
                            sem_w0)
                if t == 7:
                    @pl.when(i + 1 < NB)
                    def _():
                        pltpu.async_copy(
                            src_hbm.at[wid, pl.ds(base + 12, 4)], swin.at[1],
                            sem_w1)
                        pltpu.async_copy(
                            dst_hbm.at[wid, pl.ds(base + 12, 4)], dwin.at[1],
                            sem_w1)
            return carry

        lax.fori_loop(0, NB, body8, 0)

        pltpu.sync_copy(src_hbm.at[wid, pl.ds(K - 1, 4)], swin.at[0])
        pltpu.sync_copy(dst_hbm.at[wid, pl.ds(K - 1, 4)], dwin.at[0])
        pltpu.async_copy(g_hbm.at[swin.at[0, 0]], rows_a, sem_a).wait()
        pltpu.sync_copy(rows_a, acc.at[dwin.at[0, 0]], add=True)
        plsc.subcore_barrier()
        pltpu.sync_copy(acc.at[pl.ds(s * STRIPE, STRIPE)],
                        out_hbm.at[c, pl.ds(s * STRIPE, STRIPE)])

    return body(g, src_idx, dst_idx, zfeat)


BLK = 512


def _tc_first(degp, x_pad, W1):
    def body(degp_ref, x_ref, w_ref, dinv_ref, g_ref):
        a = degp_ref[...]
        deg = a[0, :, 0:1] + a[1, :, 0:1]
        dinv = jnp.where(deg > 0, 1.0 / jnp.sqrt(jnp.maximum(deg, 1.0)), 0.0)
        db = jnp.broadcast_to(dinv, (BLK, F))
        dinv_ref[...] = db
        g_ref[...] = db * jnp.dot(x_ref[...], w_ref[...],
                                  preferred_element_type=jnp.float32,
                                  precision=lax.Precision.HIGHEST)

    return pl.pallas_call(
        body,
        grid=(N_PAD // BLK,),
        in_specs=[
            pl.BlockSpec((2, BLK, DEGW), lambda i: (0, i, 0)),
            pl.BlockSpec((BLK, F), lambda i: (i, 0)),
            pl.BlockSpec((F, F), lambda i: (0, 0)),
        ],
        out_specs=[
            pl.BlockSpec((BLK, F), lambda i: (i, 0)),
            pl.BlockSpec((BLK, F), lambda i: (i, 0)),
        ],
        out_shape=[
            jax.ShapeDtypeStruct((N_PAD, F), jnp.float32),
            jax.ShapeDtypeStruct((N_PAD, F), jnp.float32),
        ],
    )(degp, x_pad, W1)


def _tc_mid(p, dinv_b, b, W):
    def body(p_ref, d_ref, b_ref, w_ref, g_ref):
        a = p_ref[...]
        d = d_ref[...]
        h = jnp.maximum(d * (a[0] + a[1]) + b_ref[...], 0.0)
        g_ref[...] = d * jnp.dot(h, w_ref[...],
                                 preferred_element_type=jnp.float32,
                                 precision=lax.Precision.HIGHEST)

    return pl.pallas_call(
        body,
        grid=(N_PAD // BLK,),
        in_specs=[
            pl.BlockSpec((2, BLK, F), lambda i: (0, i, 0)),
            pl.BlockSpec((BLK, F), lambda i: (i, 0)),
            pl.BlockSpec((1, F), lambda i: (0, 0)),
            pl.BlockSpec((F, F), lambda i: (0, 0)),
        ],
        out_specs=pl.BlockSpec((BLK, F), lambda i: (i, 0)),
        out_shape=jax.ShapeDtypeStruct((N_PAD, F), jnp.float32),
    )(p, dinv_b, b.reshape(1, F), W)


BLKF = 1000


def _tc_final(p, dinv_b, b3, M1, mb1, M2, mb2):
    def body(p_ref, d_ref, b_ref, m1_ref, mb1_ref, m2_ref, mb2_ref, y_ref):
        a = p_ref[...]
        h = jnp.maximum(d_ref[...] * (a[0] + a[1]) + b_ref[...], 0.0)
        u = jnp.dot(h, m1_ref[...],
                    preferred_element_type=jnp.float32,
                    precision=lax.Precision.HIGHEST) + mb1_ref[...]
        u = jnp.where(u > 0, u, jnp.exp(jnp.minimum(u, 0.0)) - 1.0)
        y = jnp.dot(u, m2_ref[...],
                    preferred_element_type=jnp.float32,
                    precision=lax.Precision.HIGHEST) + mb2_ref[...]
        y = y - jnp.max(y, axis=1, keepdims=True)
        y_ref[...] = y - jnp.log(jnp.sum(jnp.exp(y), axis=1, keepdims=True))

    return pl.pallas_call(
        body,
        grid=(N // BLKF,),
        in_specs=[
            pl.BlockSpec((2, BLKF, F), lambda i: (0, i, 0)),
            pl.BlockSpec((BLKF, F), lambda i: (i, 0)),
            pl.BlockSpec((1, F), lambda i: (0, 0)),
            pl.BlockSpec((F, 2 * F), lambda i: (0, 0)),
            pl.BlockSpec((1, 2 * F), lambda i: (0, 0)),
            pl.BlockSpec((2 * F, NLABEL), lambda i: (0, 0)),
            pl.BlockSpec((1, NLABEL), lambda i: (0, 0)),
        ],
        out_specs=pl.BlockSpec((BLKF, NLABEL), lambda i: (i, 0)),
        out_shape=jax.ShapeDtypeStruct((N, NLABEL), jnp.float32),
    )(p, dinv_b, b3.reshape(1, F), M1, mb1.reshape(1, 2 * F),
      M2, mb2.reshape(1, NLABEL))


def kernel(x, adj, W1, b1, W2, b2, W3, b3, M1, mb1, M2, mb2):
    loops = jnp.arange(N, dtype=jnp.int32)
    src = jnp.concatenate([adj[0].astype(jnp.int32), loops])
    dst = jnp.concatenate([adj[1].astype(jnp.int32), loops])
    fill = N + (jnp.arange(NW * K * C - EE, dtype=jnp.int32) % (N_PAD - N))
    src_k = jnp.concatenate([src, fill]).reshape(NW, K, C)
    dst_k = jnp.concatenate([dst, fill]).reshape(NW, K, C)
    pad_rows = jnp.full((NW, KP - K, C), N, jnp.int32)
    src_idx = jnp.concatenate([src_k, pad_rows], axis=1)
    dst_idx = jnp.concatenate([dst_k, pad_rows], axis=1)
    x_pad = jnp.zeros((N_PAD, F), jnp.float32).at[:N].set(x)
    ones = jnp.ones((C, DEGW), jnp.float32)
    zdeg = jnp.zeros((N_PAD, DEGW), jnp.float32)
    zfeat = jnp.zeros((N_PAD, F), jnp.float32)

    degp = _sc_deg(dst_idx, ones, zdeg)
    dinv_b, g = _tc_first(degp, x_pad, W1)
    p = _sc_scatter(g, src_idx, dst_idx, zfeat)
    g = _tc_mid(p, dinv_b, b1, W2)
    p = _sc_scatter(g, src_idx, dst_idx, zfeat)
    g = _tc_mid(p, dinv_b, b2, W3)
    p = _sc_scatter(g, src_idx, dst_idx, zfeat)
    return _tc_final(p, dinv_b, b3, M1, mb1, M2, mb2)
